# baseline (speedup 1.0000x reference)
import functools

import jax
import jax.numpy as jnp
from jax import lax
from jax.experimental import pallas as pl
from jax.experimental.pallas import tpu as pltpu
from jax.experimental.pallas import tpu_sc as plsc

N_NODES = 10000
N_PAD = 10000
N_EDGES = 320000
C = 128

NC = 2
NS = 16
NW = NC * NS
CH = C // NC
EPW = N_EDGES // NW
EPT = N_EDGES // NS
CHUNK = 80
DCHUNK = 40
DNCH = EPW // DCHUNK
NCHT = EPT // CHUNK
NBUF = 4
ROWS_PT = N_PAD // NS

def _deg_body(dst_hbm, ones_hbm, deg_hbm, didx, ones_v, dsem0, dsem1, acc):
    c = lax.axis_index("c")
    s = lax.axis_index("s")
    w = c * NS + s
    pltpu.sync_copy(ones_hbm, acc.at[pl.ds(s * ROWS_PT, ROWS_PT)])
    pltpu.sync_copy(dst_hbm.at[w], didx)
    pltpu.sync_copy(ones_hbm.at[pl.ds(0, DCHUNK)], ones_v)
    plsc.subcore_barrier()

    pltpu.async_copy(ones_v, acc.at[didx.at[0]], dsem0, add=True)
    pltpu.async_copy(ones_v, acc.at[didx.at[1]], dsem1, add=True)

    def body(i, carry):
        j = 2 * i + 2
        for b, sem in ((0, dsem0), (1, dsem1)):
            jj = j + b
            pltpu.make_async_copy(ones_v, acc.at[didx.at[jj]], sem).wait()
            pltpu.async_copy(ones_v, acc.at[didx.at[jj]], sem, add=True)
        return carry

    lax.fori_loop(0, DNCH // 2 - 1, body, 0)
    for sem in (dsem0, dsem1):
        pltpu.make_async_copy(ones_v, acc.at[didx.at[0]], sem).wait()
    plsc.subcore_barrier()
    pltpu.sync_copy(acc.at[pl.ds(s * ROWS_PT, ROWS_PT)],
                    deg_hbm.at[c, pl.ds(s * ROWS_PT, ROWS_PT)])


def _edge_body(hs_hbm, src_hbm, dst_hbm, out_hbm,
               sidx, didx, buf0, buf1, buf2, buf3,
               gsem0, gsem1, gsem2, gsem3,
               ssem0, ssem1, ssem2, ssem3, acc):
    c = lax.axis_index("c")
    s = lax.axis_index("s")
    hs_c = hs_hbm.at[c]
    pltpu.sync_copy(hs_c.at[pl.ds(s * ROWS_PT, ROWS_PT)],
                    acc.at[pl.ds(s * ROWS_PT, ROWS_PT)])
    pltpu.sync_copy(src_hbm.at[s], sidx)
    pltpu.sync_copy(dst_hbm.at[s], didx)
    plsc.subcore_barrier()

    bufs = (buf0, buf1, buf2, buf3)
    gsems = (gsem0, gsem1, gsem2, gsem3)
    ssems = (ssem0, ssem1, ssem2, ssem3)

    def g_start(jj, b):
        pltpu.async_copy(hs_c.at[sidx.at[jj]], bufs[b], gsems[b])

    def g_wait(jj, b):
        pltpu.make_async_copy(hs_c.at[sidx.at[jj]], bufs[b], gsems[b]).wait()

    def s_start(jj, b):
        pltpu.async_copy(bufs[b], acc.at[didx.at[jj]], ssems[b], add=True)

    def s_wait(jj, b):
        pltpu.make_async_copy(bufs[b], acc.at[didx.at[jj]], ssems[b]).wait()

    g_start(0, 0)
    g_start(1, 1)
    g_wait(0, 0)
    s_start(0, 0)
    g_start(2, 2)
    g_wait(1, 1)
    s_start(1, 1)
    g_start(3, 3)
    s_wait(0, 0)
    g_start(4, 0)
    g_wait(2, 2)
    s_start(2, 2)
    s_wait(1, 1)
    g_start(5, 1)
    g_wait(3, 3)
    s_start(3, 3)

    def body(i, carry):
        j = 4 * i + 4
        for k in range(4):
            jj = j + k
            b = k
            bn = (k + 2) % 4
            pltpu.make_async_copy(bufs[bn], acc.at[didx.at[jj]], ssems[bn]).wait()
            pltpu.async_copy(hs_c.at[sidx.at[jj + 2]], bufs[bn], gsems[bn])
            g_wait(jj, b)
            s_start(jj, b)
        return carry

    assert (NCHT - 6) % 4 == 0
    lax.fori_loop(0, (NCHT - 6) // 4, body, 0)
    for jj in range(NCHT - 2, NCHT):
        b = jj % 4
        g_wait(jj, b)
        s_start(jj, b)
    for jj in range(NCHT - 4, NCHT):
        s_wait(jj, jj % 4)
    plsc.subcore_barrier()
    pltpu.sync_copy(acc.at[pl.ds(s * ROWS_PT, ROWS_PT)],
                    out_hbm.at[c, pl.ds(s * ROWS_PT, ROWS_PT)])


@functools.cache
def _sc_kernels():
    mesh = plsc.VectorSubcoreMesh(core_axis_name="c", subcore_axis_name="s",
                                  num_cores=NC, num_subcores=NS)
    sc_params = pltpu.CompilerParams(use_tc_tiling_on_sc=False)
    deg_kernel = pl.kernel(
        _deg_body,
        compiler_params=sc_params,
        out_type=jax.ShapeDtypeStruct((NC, N_PAD, 16), jnp.float32),
        mesh=mesh,
        scratch_types=[
            pltpu.VMEM((DNCH, DCHUNK), jnp.int32),
            pltpu.VMEM((DCHUNK, 16), jnp.float32),
            pltpu.SemaphoreType.DMA,
            pltpu.SemaphoreType.DMA,
            pltpu.VMEM_SHARED((N_PAD, 16), jnp.float32),
        ],
    )
    edge_kernel = pl.kernel(
        _edge_body,
        compiler_params=sc_params,
        out_type=jax.ShapeDtypeStruct((NC, N_PAD, CH), jnp.float32),
        mesh=mesh,
        scratch_types=[
            pltpu.VMEM((NCHT, CHUNK), jnp.int32),
            pltpu.VMEM((NCHT, CHUNK), jnp.int32),
            pltpu.VMEM((CHUNK, CH), jnp.float32),
            pltpu.VMEM((CHUNK, CH), jnp.float32),
            pltpu.VMEM((CHUNK, CH), jnp.float32),
            pltpu.VMEM((CHUNK, CH), jnp.float32),
            pltpu.SemaphoreType.DMA,
            pltpu.SemaphoreType.DMA,
            pltpu.SemaphoreType.DMA,
            pltpu.SemaphoreType.DMA,
            pltpu.SemaphoreType.DMA,
            pltpu.SemaphoreType.DMA,
            pltpu.SemaphoreType.DMA,
            pltpu.SemaphoreType.DMA,
            pltpu.VMEM_SHARED((N_PAD, CH), jnp.float32),
        ],
    )
    return deg_kernel, edge_kernel


_BR = 1000
_GRID = N_PAD // _BR


def _dinv_of(deg_ref):
    d3 = deg_ref[...]
    deg = d3[0, :, 0] + d3[1, :, 0] - 1.0
    return lax.rsqrt(jnp.maximum(deg, 1.0))


def _split_store(ref, val):
    ref[0] = val[:, :CH]
    ref[1] = val[:, CH:]


def _join(ref):
    return jnp.concatenate([ref[0], ref[1]], axis=1)


def _l1_body(deg_ref, x_ref, w1_ref, hs1_ref):
    dinv = _dinv_of(deg_ref)
    h = jnp.dot(x_ref[...], w1_ref[...], preferred_element_type=jnp.float32,
                precision=lax.Precision.HIGHEST)
    _split_store(hs1_ref, h * dinv[:, None])


def _l2_body(deg_ref, t_ref, w2_ref, b1_ref, hs2_ref):
    dinv = _dinv_of(deg_ref)
    agg = _join(t_ref[...])
    z = jnp.maximum(agg * dinv[:, None] + b1_ref[...], 0.0)
    h2 = jnp.dot(z, w2_ref[...], preferred_element_type=jnp.float32,
                 precision=lax.Precision.HIGHEST)
    _split_store(hs2_ref, h2 * dinv[:, None])


def _out_body(deg_ref, t_ref, b2_ref, out_ref):
    dinv = _dinv_of(deg_ref)
    agg = _join(t_ref[...])
    out_ref[...] = agg * dinv[:, None] + b2_ref[...]


_deg_spec = pl.BlockSpec((NC, _BR, 16), lambda i: (0, i, 0))
_row_spec = pl.BlockSpec((_BR, C), lambda i: (i, 0))
_half_spec = pl.BlockSpec((NC, _BR, CH), lambda i: (0, i, 0))
_half_out = jax.ShapeDtypeStruct((NC, N_PAD, CH), jnp.float32)
_w_spec = pl.BlockSpec((C, C), lambda i: (0, 0))
_b_spec = pl.BlockSpec((1, C), lambda i: (0, 0))
_row_out = jax.ShapeDtypeStruct((N_PAD, C), jnp.float32)

_l1 = pl.pallas_call(
    _l1_body, grid=(_GRID,),
    in_specs=[_deg_spec, _row_spec, _w_spec],
    out_specs=_half_spec, out_shape=_half_out,
)
_l2 = pl.pallas_call(
    _l2_body, grid=(_GRID,),
    in_specs=[_deg_spec, _half_spec, _w_spec, _b_spec],
    out_specs=_half_spec, out_shape=_half_out,
)
_out = pl.pallas_call(
    _out_body, grid=(_GRID,),
    in_specs=[_deg_spec, _half_spec, _b_spec],
    out_specs=_row_spec, out_shape=_row_out,
)


def kernel(x, edge_index, W1, b1, W2, b2):
    _deg_kernel, _edge_kernel = _sc_kernels()
    dstd = edge_index[1].reshape(NW, DNCH, DCHUNK)
    src3 = edge_index[0].reshape(NS, NCHT, CHUNK)
    dst3 = edge_index[1].reshape(NS, NCHT, CHUNK)
    ones = jnp.ones((ROWS_PT, 16), jnp.float32)
    deg = _deg_kernel(dstd, ones)
    hs1 = _l1(deg, x, W1)
    t1 = _edge_kernel(hs1, src3, dst3)
    hs2 = _l2(deg, t1, W2, b1.reshape(1, C))
    t2 = _edge_kernel(hs2, src3, dst3)
    return _out(deg, t2, b2.reshape(1, C))


# speedup vs baseline: 1.0072x; 1.0072x over previous
import functools

import jax
import jax.numpy as jnp
from jax import lax
from jax.experimental import pallas as pl
from jax.experimental.pallas import tpu as pltpu
from jax.experimental.pallas import tpu_sc as plsc

N_NODES = 10000
N_PAD = 10000
N_EDGES = 320000
C = 128

NC = 2
NS = 16
NW = NC * NS
CH = C // NC
EPW = N_EDGES // NW
EPT = N_EDGES // NS
CHUNK = 80
DCHUNK = 40
DNCH = EPW // DCHUNK
NCHT = EPT // CHUNK
NBUF = 4
ROWS_PT = N_PAD // NS

def _deg_body(dst_hbm, ones_hbm, deg_hbm, didx, ones_v, dsem0, dsem1, acc):
    c = lax.axis_index("c")
    s = lax.axis_index("s")
    w = c * NS + s
    pltpu.sync_copy(ones_hbm, acc.at[pl.ds(s * ROWS_PT, ROWS_PT)])
    pltpu.sync_copy(dst_hbm.at[w], didx)
    pltpu.sync_copy(ones_hbm.at[pl.ds(0, DCHUNK)], ones_v)
    plsc.subcore_barrier()

    pltpu.async_copy(ones_v, acc.at[didx.at[0]], dsem0, add=True)
    pltpu.async_copy(ones_v, acc.at[didx.at[1]], dsem1, add=True)

    def body(i, carry):
        j = 2 * i + 2
        for b, sem in ((0, dsem0), (1, dsem1)):
            jj = j + b
            pltpu.make_async_copy(ones_v, acc.at[didx.at[jj]], sem).wait()
            pltpu.async_copy(ones_v, acc.at[didx.at[jj]], sem, add=True)
        return carry

    lax.fori_loop(0, DNCH // 2 - 1, body, 0)
    for sem in (dsem0, dsem1):
        pltpu.make_async_copy(ones_v, acc.at[didx.at[0]], sem).wait()
    plsc.subcore_barrier()
    pltpu.sync_copy(acc.at[pl.ds(s * ROWS_PT, ROWS_PT)],
                    deg_hbm.at[c, pl.ds(s * ROWS_PT, ROWS_PT)])


def _edge_body(hs_hbm, src_hbm, dst_hbm, out_hbm,
               sidx, didx, buf0, buf1, buf2, buf3,
               gsem0, gsem1, gsem2, gsem3,
               ssem0, ssem1, ssem2, ssem3, acc):
    c = lax.axis_index("c")
    s = lax.axis_index("s")
    hs_c = hs_hbm.at[c]
    pltpu.sync_copy(hs_c.at[pl.ds(s * ROWS_PT, ROWS_PT)],
                    acc.at[pl.ds(s * ROWS_PT, ROWS_PT)])
    pltpu.sync_copy(src_hbm.at[s], sidx)
    pltpu.sync_copy(dst_hbm.at[s], didx)
    plsc.subcore_barrier()

    bufs = (buf0, buf1, buf2, buf3)
    gsems = (gsem0, gsem1, gsem2, gsem3)
    ssems = (ssem0, ssem1, ssem2, ssem3)

    def g_start(jj, b):
        pltpu.async_copy(hs_c.at[sidx.at[jj]], bufs[b], gsems[b])

    def g_wait(jj, b):
        pltpu.make_async_copy(hs_c.at[sidx.at[jj]], bufs[b], gsems[b]).wait()

    def s_start(jj, b):
        pltpu.async_copy(bufs[b], acc.at[didx.at[jj]], ssems[b], add=True)

    def s_wait(jj, b):
        pltpu.make_async_copy(bufs[b], acc.at[didx.at[jj]], ssems[b]).wait()

    g_start(0, 0)
    g_start(1, 1)
    g_wait(0, 0)
    s_start(0, 0)
    g_start(2, 2)
    g_wait(1, 1)
    s_start(1, 1)
    g_start(3, 3)
    s_wait(0, 0)
    g_start(4, 0)
    g_wait(2, 2)
    s_start(2, 2)
    s_wait(1, 1)
    g_start(5, 1)
    g_wait(3, 3)
    s_start(3, 3)

    def body(i, carry):
        j = 4 * i + 4
        for k in range(4):
            jj = j + k
            b = k
            bn = (k + 2) % 4
            pltpu.make_async_copy(bufs[bn], acc.at[didx.at[jj]], ssems[bn]).wait()
            pltpu.async_copy(hs_c.at[sidx.at[jj + 2]], bufs[bn], gsems[bn])
            g_wait(jj, b)
            s_start(jj, b)
        return carry

    assert (NCHT - 6) % 4 == 0
    lax.fori_loop(0, (NCHT - 6) // 4, body, 0)
    for jj in range(NCHT - 2, NCHT):
        b = jj % 4
        g_wait(jj, b)
        s_start(jj, b)
    for jj in range(NCHT - 4, NCHT):
        s_wait(jj, jj % 4)
    plsc.subcore_barrier()
    pltpu.sync_copy(acc.at[pl.ds(s * ROWS_PT, ROWS_PT)],
                    out_hbm.at[c, pl.ds(s * ROWS_PT, ROWS_PT)])


@functools.cache
def _sc_kernels():
    mesh = plsc.VectorSubcoreMesh(core_axis_name="c", subcore_axis_name="s",
                                  num_cores=NC, num_subcores=NS)
    sc_params = pltpu.CompilerParams(use_tc_tiling_on_sc=False)
    deg_kernel = pl.kernel(
        _deg_body,
        compiler_params=sc_params,
        out_type=jax.ShapeDtypeStruct((NC, N_PAD, 16), jnp.float32),
        mesh=mesh,
        scratch_types=[
            pltpu.VMEM((DNCH, DCHUNK), jnp.int32),
            pltpu.VMEM((DCHUNK, 16), jnp.float32),
            pltpu.SemaphoreType.DMA,
            pltpu.SemaphoreType.DMA,
            pltpu.VMEM_SHARED((N_PAD, 16), jnp.float32),
        ],
    )
    edge_kernel = pl.kernel(
        _edge_body,
        compiler_params=sc_params,
        out_type=jax.ShapeDtypeStruct((NC, N_PAD, CH), jnp.float32),
        mesh=mesh,
        scratch_types=[
            pltpu.VMEM((NCHT, CHUNK), jnp.int32),
            pltpu.VMEM((NCHT, CHUNK), jnp.int32),
            pltpu.VMEM((CHUNK, CH), jnp.float32),
            pltpu.VMEM((CHUNK, CH), jnp.float32),
            pltpu.VMEM((CHUNK, CH), jnp.float32),
            pltpu.VMEM((CHUNK, CH), jnp.float32),
            pltpu.SemaphoreType.DMA,
            pltpu.SemaphoreType.DMA,
            pltpu.SemaphoreType.DMA,
            pltpu.SemaphoreType.DMA,
            pltpu.SemaphoreType.DMA,
            pltpu.SemaphoreType.DMA,
            pltpu.SemaphoreType.DMA,
            pltpu.SemaphoreType.DMA,
            pltpu.VMEM_SHARED((N_PAD, CH), jnp.float32),
        ],
    )
    return deg_kernel, edge_kernel


_BR = 1000
_GRID = N_PAD // _BR


def _dinv_of(deg_ref):
    d3 = deg_ref[...]
    deg = d3[0, :, 0] + d3[1, :, 0] - 1.0
    return lax.rsqrt(jnp.maximum(deg, 1.0))


def _split_store(ref, val):
    ref[0] = val[:, :CH]
    ref[1] = val[:, CH:]


def _join(ref):
    return jnp.concatenate([ref[0], ref[1]], axis=1)


def _mm1_body(x_ref, w1_ref, h1_ref):
    h1_ref[...] = jnp.dot(x_ref[...], w1_ref[...],
                          preferred_element_type=jnp.float32,
                          precision=lax.Precision.HIGHEST)


def _scale1_body(deg_ref, h1_ref, hs1_ref):
    dinv = _dinv_of(deg_ref)
    _split_store(hs1_ref, h1_ref[...] * dinv[:, None])


def _l2_body(deg_ref, t_ref, w2_ref, b1_ref, hs2_ref):
    dinv = _dinv_of(deg_ref)
    agg = _join(t_ref[...])
    z = jnp.maximum(agg * dinv[:, None] + b1_ref[...], 0.0)
    h2 = jnp.dot(z, w2_ref[...], preferred_element_type=jnp.float32,
                 precision=lax.Precision.HIGHEST)
    _split_store(hs2_ref, h2 * dinv[:, None])


def _out_body(deg_ref, t_ref, b2_ref, out_ref):
    dinv = _dinv_of(deg_ref)
    agg = _join(t_ref[...])
    out_ref[...] = agg * dinv[:, None] + b2_ref[...]


_deg_spec = pl.BlockSpec((NC, _BR, 16), lambda i: (0, i, 0))
_row_spec = pl.BlockSpec((_BR, C), lambda i: (i, 0))
_half_spec = pl.BlockSpec((NC, _BR, CH), lambda i: (0, i, 0))
_half_out = jax.ShapeDtypeStruct((NC, N_PAD, CH), jnp.float32)
_w_spec = pl.BlockSpec((C, C), lambda i: (0, 0))
_b_spec = pl.BlockSpec((1, C), lambda i: (0, 0))
_row_out = jax.ShapeDtypeStruct((N_PAD, C), jnp.float32)

_mm1 = pl.pallas_call(
    _mm1_body, grid=(_GRID,),
    in_specs=[_row_spec, _w_spec],
    out_specs=_row_spec, out_shape=_row_out,
)
_scale1 = pl.pallas_call(
    _scale1_body, grid=(_GRID,),
    in_specs=[_deg_spec, _row_spec],
    out_specs=_half_spec, out_shape=_half_out,
)
_l2 = pl.pallas_call(
    _l2_body, grid=(_GRID,),
    in_specs=[_deg_spec, _half_spec, _w_spec, _b_spec],
    out_specs=_half_spec, out_shape=_half_out,
)
_out = pl.pallas_call(
    _out_body, grid=(_GRID,),
    in_specs=[_deg_spec, _half_spec, _b_spec],
    out_specs=_row_spec, out_shape=_row_out,
)


def kernel(x, edge_index, W1, b1, W2, b2):
    _deg_kernel, _edge_kernel = _sc_kernels()
    dstd = edge_index[1].reshape(NW, DNCH, DCHUNK)
    src3 = edge_index[0].reshape(NS, NCHT, CHUNK)
    dst3 = edge_index[1].reshape(NS, NCHT, CHUNK)
    ones = jnp.ones((ROWS_PT, 16), jnp.float32)
    h1 = _mm1(x, W1)
    deg = _deg_kernel(dstd, ones)
    hs1 = _scale1(deg, h1)
    t1 = _edge_kernel(hs1, src3, dst3)
    hs2 = _l2(deg, t1, W2, b1.reshape(1, C))
    t2 = _edge_kernel(hs2, src3, dst3)
    return _out(deg, t2, b2.reshape(1, C))


# speedup vs baseline: 1.0092x; 1.0020x over previous
import functools

import jax
import jax.numpy as jnp
from jax import lax
from jax.experimental import pallas as pl
from jax.experimental.pallas import tpu as pltpu
from jax.experimental.pallas import tpu_sc as plsc

N_NODES = 10000
N_PAD = 10000
N_EDGES = 320000
C = 128

NC = 2
NS = 16
NW = NC * NS
CH = C // NC
EPW = N_EDGES // NW
EPT = N_EDGES // NS
CHUNK = 100
DCHUNK = 40
DNCH = EPW // DCHUNK
NCHT = EPT // CHUNK
NBUF = 4
ROWS_PT = N_PAD // NS

def _deg_body(dst_hbm, ones_hbm, deg_hbm, didx, ones_v, dsem0, dsem1, acc):
    c = lax.axis_index("c")
    s = lax.axis_index("s")
    w = c * NS + s
    pltpu.sync_copy(ones_hbm, acc.at[pl.ds(s * ROWS_PT, ROWS_PT)])
    pltpu.sync_copy(dst_hbm.at[w], didx)
    pltpu.sync_copy(ones_hbm.at[pl.ds(0, DCHUNK)], ones_v)
    plsc.subcore_barrier()

    pltpu.async_copy(ones_v, acc.at[didx.at[0]], dsem0, add=True)
    pltpu.async_copy(ones_v, acc.at[didx.at[1]], dsem1, add=True)

    def body(i, carry):
        j = 2 * i + 2
        for b, sem in ((0, dsem0), (1, dsem1)):
            jj = j + b
            pltpu.make_async_copy(ones_v, acc.at[didx.at[jj]], sem).wait()
            pltpu.async_copy(ones_v, acc.at[didx.at[jj]], sem, add=True)
        return carry

    lax.fori_loop(0, DNCH // 2 - 1, body, 0)
    for sem in (dsem0, dsem1):
        pltpu.make_async_copy(ones_v, acc.at[didx.at[0]], sem).wait()
    plsc.subcore_barrier()
    pltpu.sync_copy(acc.at[pl.ds(s * ROWS_PT, ROWS_PT)],
                    deg_hbm.at[c, pl.ds(s * ROWS_PT, ROWS_PT)])


def _edge_body(hs_hbm, src_hbm, dst_hbm, out_hbm,
               sidx, didx, buf0, buf1, buf2, buf3,
               gsem0, gsem1, gsem2, gsem3,
               ssem0, ssem1, ssem2, ssem3, acc):
    c = lax.axis_index("c")
    s = lax.axis_index("s")
    hs_c = hs_hbm.at[c]
    pltpu.sync_copy(hs_c.at[pl.ds(s * ROWS_PT, ROWS_PT)],
                    acc.at[pl.ds(s * ROWS_PT, ROWS_PT)])
    pltpu.sync_copy(src_hbm.at[s], sidx)
    pltpu.sync_copy(dst_hbm.at[s], didx)
    plsc.subcore_barrier()

    bufs = (buf0, buf1, buf2, buf3)
    gsems = (gsem0, gsem1, gsem2, gsem3)
    ssems = (ssem0, ssem1, ssem2, ssem3)

    def g_start(jj, b):
        pltpu.async_copy(hs_c.at[sidx.at[jj]], bufs[b], gsems[b])

    def g_wait(jj, b):
        pltpu.make_async_copy(hs_c.at[sidx.at[jj]], bufs[b], gsems[b]).wait()

    def s_start(jj, b):
        pltpu.async_copy(bufs[b], acc.at[didx.at[jj]], ssems[b], add=True)

    def s_wait(jj, b):
        pltpu.make_async_copy(bufs[b], acc.at[didx.at[jj]], ssems[b]).wait()

    g_start(0, 0)
    g_start(1, 1)
    g_wait(0, 0)
    s_start(0, 0)
    g_start(2, 2)
    g_wait(1, 1)
    s_start(1, 1)
    g_start(3, 3)
    s_wait(0, 0)
    g_start(4, 0)
    g_wait(2, 2)
    s_start(2, 2)
    s_wait(1, 1)
    g_start(5, 1)
    g_wait(3, 3)
    s_start(3, 3)

    def body(i, carry):
        j = 4 * i + 6
        for k in range(4):
            jj = j + k
            b = (k + 2) % 4
            bn = k % 4
            pltpu.make_async_copy(bufs[bn], acc.at[didx.at[jj]], ssems[bn]).wait()
            pltpu.async_copy(hs_c.at[sidx.at[jj + 2]], bufs[bn], gsems[bn])
            g_wait(jj, b)
            s_start(jj, b)
        return carry

    for jj in (4, 5):
        b = jj % 4
        bn = (jj + 2) % 4
        s_wait(jj - 2, bn)
        g_start(jj + 2, bn)
        g_wait(jj, b)
        s_start(jj, b)
    assert (NCHT - 8) % 4 == 0
    lax.fori_loop(0, (NCHT - 8) // 4, body, 0)
    for jj in range(NCHT - 2, NCHT):
        b = jj % 4
        g_wait(jj, b)
        s_start(jj, b)
    for jj in range(NCHT - 4, NCHT):
        s_wait(jj, jj % 4)
    plsc.subcore_barrier()
    pltpu.sync_copy(acc.at[pl.ds(s * ROWS_PT, ROWS_PT)],
                    out_hbm.at[c, pl.ds(s * ROWS_PT, ROWS_PT)])


@functools.cache
def _sc_kernels():
    mesh = plsc.VectorSubcoreMesh(core_axis_name="c", subcore_axis_name="s",
                                  num_cores=NC, num_subcores=NS)
    sc_params = pltpu.CompilerParams(use_tc_tiling_on_sc=False)
    deg_kernel = pl.kernel(
        _deg_body,
        compiler_params=sc_params,
        out_type=jax.ShapeDtypeStruct((NC, N_PAD, 16), jnp.float32),
        mesh=mesh,
        scratch_types=[
            pltpu.VMEM((DNCH, DCHUNK), jnp.int32),
            pltpu.VMEM((DCHUNK, 16), jnp.float32),
            pltpu.SemaphoreType.DMA,
            pltpu.SemaphoreType.DMA,
            pltpu.VMEM_SHARED((N_PAD, 16), jnp.float32),
        ],
    )
    edge_kernel = pl.kernel(
        _edge_body,
        compiler_params=sc_params,
        out_type=jax.ShapeDtypeStruct((NC, N_PAD, CH), jnp.float32),
        mesh=mesh,
        scratch_types=[
            pltpu.VMEM((NCHT, CHUNK), jnp.int32),
            pltpu.VMEM((NCHT, CHUNK), jnp.int32),
            pltpu.VMEM((CHUNK, CH), jnp.float32),
            pltpu.VMEM((CHUNK, CH), jnp.float32),
            pltpu.VMEM((CHUNK, CH), jnp.float32),
            pltpu.VMEM((CHUNK, CH), jnp.float32),
            pltpu.SemaphoreType.DMA,
            pltpu.SemaphoreType.DMA,
            pltpu.SemaphoreType.DMA,
            pltpu.SemaphoreType.DMA,
            pltpu.SemaphoreType.DMA,
            pltpu.SemaphoreType.DMA,
            pltpu.SemaphoreType.DMA,
            pltpu.SemaphoreType.DMA,
            pltpu.VMEM_SHARED((N_PAD, CH), jnp.float32),
        ],
    )
    return deg_kernel, edge_kernel


_BR = 1000
_GRID = N_PAD // _BR


def _dinv_of(deg_ref):
    d3 = deg_ref[...]
    deg = d3[0, :, 0] + d3[1, :, 0] - 1.0
    return lax.rsqrt(jnp.maximum(deg, 1.0))


def _split_store(ref, val):
    ref[0] = val[:, :CH]
    ref[1] = val[:, CH:]


def _join(ref):
    return jnp.concatenate([ref[0], ref[1]], axis=1)


def _mm1_body(x_ref, w1_ref, h1_ref):
    h1_ref[...] = jnp.dot(x_ref[...], w1_ref[...],
                          preferred_element_type=jnp.float32,
                          precision=lax.Precision.HIGHEST)


def _scale1_body(deg_ref, h1_ref, hs1_ref):
    dinv = _dinv_of(deg_ref)
    _split_store(hs1_ref, h1_ref[...] * dinv[:, None])


def _l2_body(deg_ref, t_ref, w2_ref, b1_ref, hs2_ref):
    dinv = _dinv_of(deg_ref)
    agg = _join(t_ref[...])
    z = jnp.maximum(agg * dinv[:, None] + b1_ref[...], 0.0)
    h2 = jnp.dot(z, w2_ref[...], preferred_element_type=jnp.float32,
                 precision=lax.Precision.HIGHEST)
    _split_store(hs2_ref, h2 * dinv[:, None])


def _out_body(deg_ref, t_ref, b2_ref, out_ref):
    dinv = _dinv_of(deg_ref)
    agg = _join(t_ref[...])
    out_ref[...] = agg * dinv[:, None] + b2_ref[...]


_deg_spec = pl.BlockSpec((NC, _BR, 16), lambda i: (0, i, 0))
_row_spec = pl.BlockSpec((_BR, C), lambda i: (i, 0))
_half_spec = pl.BlockSpec((NC, _BR, CH), lambda i: (0, i, 0))
_half_out = jax.ShapeDtypeStruct((NC, N_PAD, CH), jnp.float32)
_w_spec = pl.BlockSpec((C, C), lambda i: (0, 0))
_b_spec = pl.BlockSpec((1, C), lambda i: (0, 0))
_row_out = jax.ShapeDtypeStruct((N_PAD, C), jnp.float32)

_mm1 = pl.pallas_call(
    _mm1_body, grid=(_GRID,),
    in_specs=[_row_spec, _w_spec],
    out_specs=_row_spec, out_shape=_row_out,
)
_scale1 = pl.pallas_call(
    _scale1_body, grid=(_GRID,),
    in_specs=[_deg_spec, _row_spec],
    out_specs=_half_spec, out_shape=_half_out,
)
_l2 = pl.pallas_call(
    _l2_body, grid=(_GRID,),
    in_specs=[_deg_spec, _half_spec, _w_spec, _b_spec],
    out_specs=_half_spec, out_shape=_half_out,
)
_out = pl.pallas_call(
    _out_body, grid=(_GRID,),
    in_specs=[_deg_spec, _half_spec, _b_spec],
    out_specs=_row_spec, out_shape=_row_out,
)


def kernel(x, edge_index, W1, b1, W2, b2):
    _deg_kernel, _edge_kernel = _sc_kernels()
    dstd = edge_index[1].reshape(NW, DNCH, DCHUNK)
    src3 = edge_index[0].reshape(NS, NCHT, CHUNK)
    dst3 = edge_index[1].reshape(NS, NCHT, CHUNK)
    ones = jnp.ones((ROWS_PT, 16), jnp.float32)
    h1 = _mm1(x, W1)
    deg = _deg_kernel(dstd, ones)
    hs1 = _scale1(deg, h1)
    t1 = _edge_kernel(hs1, src3, dst3)
    hs2 = _l2(deg, t1, W2, b1.reshape(1, C))
    t2 = _edge_kernel(hs2, src3, dst3)
    return _out(deg, t2, b2.reshape(1, C))
